# baseline (speedup 1.0000x reference)
import functools

import jax
import jax.numpy as jnp
from jax.experimental import pallas as pl

_EPS = 1e-12
_VP = 512


def _body(x_ref, ie_ref, io_ref, tab_ref, m_ref, g_ref, b_ref, o_ref, *, rows, vp):
    idx_e = ie_ref[0, 0, :]
    idx_o = io_ref[0, 0, :]
    iota = jax.lax.broadcasted_iota(jnp.int32, (rows, vp), 1)
    oh_e = (idx_e[:, None] == iota).astype(jnp.bfloat16)
    oh_o = (idx_o[:, None] == iota).astype(jnp.bfloat16)
    dims = (((1,), (0,)), ((), ()))
    pe_e = jax.lax.dot_general(oh_e, tab_ref[...], dimension_numbers=dims,
                               preferred_element_type=jnp.float32)
    pe_o = jax.lax.dot_general(oh_o, tab_ref[...], dimension_numbers=dims,
                               preferred_element_type=jnp.float32)
    pe = jnp.concatenate((pe_e, pe_o), axis=1)
    h = x_ref[...] + pe
    hb = h.astype(jnp.bfloat16)
    mean = jax.lax.dot_general(hb, m_ref[...], dimension_numbers=dims,
                               preferred_element_type=jnp.float32)
    c = h - mean
    cb = (c * c).astype(jnp.bfloat16)
    var = jax.lax.dot_general(cb, m_ref[...], dimension_numbers=dims,
                              preferred_element_type=jnp.float32)
    o_ref[...] = (c * jax.lax.rsqrt(var + _EPS)) * g_ref[...] + b_ref[...]


def kernel(x, position_tags, emb_table, gamma, beta):
    b, l, f = x.shape
    n = b * l
    n2 = n // 2
    f2 = 2 * f
    rows = 2048 if n2 % 2048 == 0 else n2
    nb = n2 // rows
    x2 = x.reshape(n2, f2)
    idx = position_tags.astype(jnp.int32).reshape(-1)
    idx_e = idx[0::2].reshape(nb, 1, rows)
    idx_o = idx[1::2].reshape(nb, 1, rows)
    tab = jnp.pad(
        emb_table, ((0, _VP - emb_table.shape[0]), (0, 0))
    ).astype(jnp.bfloat16)
    half = jnp.arange(f2) >= f
    mavg = jnp.where(half[:, None] == half[None, :], 1.0 / f, 0.0).astype(
        jnp.bfloat16
    )
    g2 = jnp.concatenate((gamma, gamma)).reshape(1, f2)
    b2 = jnp.concatenate((beta, beta)).reshape(1, f2)
    out = pl.pallas_call(
        functools.partial(_body, rows=rows, vp=_VP),
        grid=(nb,),
        in_specs=[
            pl.BlockSpec((rows, f2), lambda i: (i, 0)),
            pl.BlockSpec((1, 1, rows), lambda i: (i, 0, 0)),
            pl.BlockSpec((1, 1, rows), lambda i: (i, 0, 0)),
            pl.BlockSpec((_VP, f), lambda i: (0, 0)),
            pl.BlockSpec((f2, f2), lambda i: (0, 0)),
            pl.BlockSpec((1, f2), lambda i: (0, 0)),
            pl.BlockSpec((1, f2), lambda i: (0, 0)),
        ],
        out_specs=pl.BlockSpec((rows, f2), lambda i: (i, 0)),
        out_shape=jax.ShapeDtypeStruct((n2, f2), jnp.float32),
    )(x2, idx_e, idx_o, tab, mavg, g2, b2)
    return out.reshape(b, l, f)


# speedup vs baseline: 2.0771x; 2.0771x over previous
import functools

import jax
import jax.numpy as jnp
from jax.experimental import pallas as pl

_EPS = 1e-12
_VP = 512


def _body(x_ref, idx_ref, tab_ref, g_ref, b_ref, o_ref, *, f, nb):
    idx = idx_ref[0, 0, :]
    xt = x_ref[0]
    idx2 = jnp.broadcast_to(idx[None, :], (f, nb))
    lo = idx2 & 127
    hi = idx2 >> 7
    pe = jnp.take_along_axis(tab_ref[:, 0:128], lo, axis=1)
    for j in range(1, _VP // 128):
        gj = jnp.take_along_axis(tab_ref[:, j * 128:(j + 1) * 128], lo, axis=1)
        pe = jnp.where(hi == j, gj, pe)
    h = xt + pe
    mean = jnp.mean(h, axis=0, keepdims=True)
    c = h - mean
    var = jnp.mean(c * c, axis=0, keepdims=True)
    o_ref[0] = (c * jax.lax.rsqrt(var + _EPS)) * g_ref[...] + b_ref[...]


def kernel(x, position_tags, emb_table, gamma, beta):
    b, l, f = x.shape
    v = emb_table.shape[0]
    xt = jnp.transpose(x, (1, 2, 0))
    idx_t = position_tags.astype(jnp.int32).T.reshape(l, 1, b)
    tab_t = jnp.pad(emb_table, ((0, _VP - v), (0, 0))).T
    g_c = gamma.reshape(f, 1)
    b_c = beta.reshape(f, 1)
    out_t = pl.pallas_call(
        functools.partial(_body, f=f, nb=b),
        grid=(l,),
        in_specs=[
            pl.BlockSpec((1, f, b), lambda i: (i, 0, 0)),
            pl.BlockSpec((1, 1, b), lambda i: (i, 0, 0)),
            pl.BlockSpec((f, _VP), lambda i: (0, 0)),
            pl.BlockSpec((f, 1), lambda i: (0, 0)),
            pl.BlockSpec((f, 1), lambda i: (0, 0)),
        ],
        out_specs=pl.BlockSpec((1, f, b), lambda i: (i, 0, 0)),
        out_shape=jax.ShapeDtypeStruct((l, f, b), jnp.float32),
    )(xt, idx_t, tab_t, g_c, b_c)
    return jnp.transpose(out_t, (2, 0, 1))


# speedup vs baseline: 2.1235x; 1.0224x over previous
import functools

import jax
import jax.numpy as jnp
from jax.experimental import pallas as pl

_EPS = 1e-12
_VP = 512


def _body(x_ref, idx_ref, w01_ref, w23_ref, g_ref, b_ref, o_ref, *, f, nb):
    idx = idx_ref[0, 0, :]
    xt = x_ref[0]
    lo = jnp.broadcast_to((idx & 127)[None, :], (f, nb))
    hi = jnp.broadcast_to((idx >> 7)[None, :], (f, nb))
    w01 = jnp.take_along_axis(w01_ref[...], lo, axis=1)
    w23 = jnp.take_along_axis(w23_ref[...], lo, axis=1)
    w = jnp.where(hi < 2, w01, w23)
    bits = jnp.where((hi & 1) == 0, w & jnp.int32(-65536), w << 16)
    pe = jax.lax.bitcast_convert_type(bits, jnp.float32)
    h = xt + pe
    mean = jnp.mean(h, axis=0, keepdims=True)
    c = h - mean
    var = jnp.mean(c * c, axis=0, keepdims=True)
    o_ref[0] = (c * jax.lax.rsqrt(var + _EPS)) * g_ref[...] + b_ref[...]


def kernel(x, position_tags, emb_table, gamma, beta):
    b, l, f = x.shape
    v = emb_table.shape[0]
    xt = jnp.transpose(x, (1, 2, 0))
    idx_t = position_tags.astype(jnp.int32).T.reshape(l, 1, b)
    tab_t = jnp.pad(emb_table, ((0, _VP - v), (0, 0))).T
    u16 = jax.lax.bitcast_convert_type(
        tab_t.astype(jnp.bfloat16), jnp.uint16
    ).astype(jnp.uint32)
    w01 = ((u16[:, 0:128] << 16) | u16[:, 128:256]).astype(jnp.int32)
    w23 = ((u16[:, 256:384] << 16) | u16[:, 384:512]).astype(jnp.int32)
    g_c = gamma.reshape(f, 1)
    b_c = beta.reshape(f, 1)
    bb = 1024 if b % 1024 == 0 else b
    out_t = pl.pallas_call(
        functools.partial(_body, f=f, nb=bb),
        grid=(l, b // bb),
        in_specs=[
            pl.BlockSpec((1, f, bb), lambda i, j: (i, 0, j)),
            pl.BlockSpec((1, 1, bb), lambda i, j: (i, 0, j)),
            pl.BlockSpec((f, 128), lambda i, j: (0, 0)),
            pl.BlockSpec((f, 128), lambda i, j: (0, 0)),
            pl.BlockSpec((f, 1), lambda i, j: (0, 0)),
            pl.BlockSpec((f, 1), lambda i, j: (0, 0)),
        ],
        out_specs=pl.BlockSpec((1, f, bb), lambda i, j: (i, 0, j)),
        out_shape=jax.ShapeDtypeStruct((l, f, b), jnp.float32),
    )(xt, idx_t, w01, w23, g_c, b_c)
    return jnp.transpose(out_t, (2, 0, 1))


# speedup vs baseline: 3.7844x; 1.7821x over previous
import functools

import jax
import jax.numpy as jnp
from jax.experimental import pallas as pl

_EPS = 1e-12
_VP = 512


def _body(x_ref, idx_ref, w01_ref, w23_ref, g_ref, b_ref, o_ref, *, f, nb):
    idx = idx_ref[0, 0, :]
    xt = x_ref[0]
    lo = jnp.broadcast_to((idx & 127)[None, :], (f, nb))
    hi = jnp.broadcast_to((idx >> 7)[None, :], (f, nb))
    w01 = jnp.take_along_axis(w01_ref[...], lo, axis=1)
    w23 = jnp.take_along_axis(w23_ref[...], lo, axis=1)
    w = jnp.where(hi < 2, w01, w23)
    bits = jnp.where((hi & 1) == 0, w & jnp.int32(-65536), w << 16)
    pe = jax.lax.bitcast_convert_type(bits, jnp.float32)
    h = xt + pe
    mean = jnp.mean(h, axis=0, keepdims=True)
    c = h - mean
    var = jnp.mean(c * c, axis=0, keepdims=True)
    o_ref[0] = (c * jax.lax.rsqrt(var + _EPS)) * g_ref[...] + b_ref[...]


def kernel(x, position_tags, emb_table, gamma, beta):
    b, l, f = x.shape
    v = emb_table.shape[0]
    xt = jnp.transpose(x, (1, 2, 0))
    idx_t = position_tags.astype(jnp.int32).T.reshape(l, 1, b)
    tab_t = jnp.pad(emb_table, ((0, _VP - v), (0, 0))).T
    u16 = jax.lax.bitcast_convert_type(
        tab_t.astype(jnp.bfloat16), jnp.uint16
    ).astype(jnp.uint32)
    w01 = ((u16[:, 0:128] << 16) | u16[:, 128:256]).astype(jnp.int32)
    w23 = ((u16[:, 256:384] << 16) | u16[:, 384:512]).astype(jnp.int32)
    g_c = gamma.reshape(f, 1)
    b_c = beta.reshape(f, 1)
    bb = 4096 if b % 4096 == 0 else b
    out_t = pl.pallas_call(
        functools.partial(_body, f=f, nb=bb),
        grid=(l, b // bb),
        in_specs=[
            pl.BlockSpec((1, f, bb), lambda i, j: (i, 0, j)),
            pl.BlockSpec((1, 1, bb), lambda i, j: (i, 0, j)),
            pl.BlockSpec((f, 128), lambda i, j: (0, 0)),
            pl.BlockSpec((f, 128), lambda i, j: (0, 0)),
            pl.BlockSpec((f, 1), lambda i, j: (0, 0)),
            pl.BlockSpec((f, 1), lambda i, j: (0, 0)),
        ],
        out_specs=pl.BlockSpec((1, f, bb), lambda i, j: (i, 0, j)),
        out_shape=jax.ShapeDtypeStruct((l, f, b), jnp.float32),
    )(xt, idx_t, w01, w23, g_c, b_c)
    return jnp.transpose(out_t, (2, 0, 1))


# speedup vs baseline: 4.0604x; 1.0729x over previous
import functools

import jax
import jax.numpy as jnp
from jax.experimental import pallas as pl

_EPS = 1e-12
_VP = 512


def _body(x_ref, idx_ref, w01_ref, w23_ref, g_ref, b_ref, o_ref, *, f, nb, lb):
    w01_t = w01_ref[...]
    w23_t = w23_ref[...]
    g_c = g_ref[...]
    b_c = b_ref[...]
    for t in range(lb):
        idx = idx_ref[t, 0, :]
        xt = x_ref[t]
        lo = jnp.broadcast_to((idx & 127)[None, :], (f, nb))
        hi2 = jnp.broadcast_to((idx >> 8)[None, :], (f, nb))
        sh = jnp.broadcast_to(((idx & 128) >> 3)[None, :], (f, nb))
        w01 = jnp.take_along_axis(w01_t, lo, axis=1)
        w23 = jnp.take_along_axis(w23_t, lo, axis=1)
        w = jnp.where(hi2 == 0, w01, w23)
        bits = (w << sh) & jnp.int32(-65536)
        pe = jax.lax.bitcast_convert_type(bits, jnp.float32)
        h = xt + pe
        mean = jnp.mean(h, axis=0, keepdims=True)
        c = h - mean
        var = jnp.mean(c * c, axis=0, keepdims=True)
        o_ref[t] = (c * jax.lax.rsqrt(var + _EPS)) * g_c + b_c


def kernel(x, position_tags, emb_table, gamma, beta):
    b, l, f = x.shape
    v = emb_table.shape[0]
    xt = jnp.transpose(x, (1, 2, 0))
    idx_t = position_tags.astype(jnp.int32).T.reshape(l, 1, b)
    tab_t = jnp.pad(emb_table, ((0, _VP - v), (0, 0))).T
    u16 = jax.lax.bitcast_convert_type(
        tab_t.astype(jnp.bfloat16), jnp.uint16
    ).astype(jnp.uint32)
    w01 = ((u16[:, 0:128] << 16) | u16[:, 128:256]).astype(jnp.int32)
    w23 = ((u16[:, 256:384] << 16) | u16[:, 384:512]).astype(jnp.int32)
    g_c = gamma.reshape(f, 1)
    b_c = beta.reshape(f, 1)
    bb = 4096 if b % 4096 == 0 else b
    lb = 4 if l % 4 == 0 else 1
    out_t = pl.pallas_call(
        functools.partial(_body, f=f, nb=bb, lb=lb),
        grid=(l // lb, b // bb),
        in_specs=[
            pl.BlockSpec((lb, f, bb), lambda i, j: (i, 0, j)),
            pl.BlockSpec((lb, 1, bb), lambda i, j: (i, 0, j)),
            pl.BlockSpec((f, 128), lambda i, j: (0, 0)),
            pl.BlockSpec((f, 128), lambda i, j: (0, 0)),
            pl.BlockSpec((f, 1), lambda i, j: (0, 0)),
            pl.BlockSpec((f, 1), lambda i, j: (0, 0)),
        ],
        out_specs=pl.BlockSpec((lb, f, bb), lambda i, j: (i, 0, j)),
        out_shape=jax.ShapeDtypeStruct((l, f, b), jnp.float32),
    )(xt, idx_t, w01, w23, g_c, b_c)
    return jnp.transpose(out_t, (2, 0, 1))


# speedup vs baseline: 5.7174x; 1.4081x over previous
import functools

import jax
import jax.numpy as jnp
from jax.experimental import pallas as pl

_EPS = 1e-12
_VP = 512


def _body(x_ref, idx_ref, w01_ref, w23_ref, g_ref, b_ref, o_ref, *, f, nb, lb, ch):
    w01_t = w01_ref[...]
    w23_t = w23_ref[...]
    g_c = g_ref[...]
    b_c = b_ref[...]
    for t in range(lb):
        for c in range(nb // ch):
            sl = pl.ds(c * ch, ch)
            idxc = idx_ref[t, 0, sl]
            lo = jnp.broadcast_to((idxc & 127)[None, :], (f, ch))
            hi2 = (idxc >> 8)[None, :]
            sh = ((idxc & 128) >> 3)[None, :]
            w01 = jnp.take_along_axis(w01_t, lo, axis=1)
            w23 = jnp.take_along_axis(w23_t, lo, axis=1)
            w = jnp.where(hi2 == 0, w01, w23)
            bits = (w << sh) & jnp.int32(-65536)
            pe = jax.lax.bitcast_convert_type(bits, jnp.float32)
            h = x_ref[t, :, sl] + pe
            mean = jnp.mean(h, axis=0, keepdims=True)
            cdev = h - mean
            var = jnp.mean(cdev * cdev, axis=0, keepdims=True)
            o_ref[t, :, sl] = (cdev * jax.lax.rsqrt(var + _EPS)) * g_c + b_c


def kernel(x, position_tags, emb_table, gamma, beta):
    b, l, f = x.shape
    v = emb_table.shape[0]
    xt = jnp.transpose(x, (1, 2, 0))
    idx_t = position_tags.astype(jnp.int32).T.reshape(l, 1, b)
    tab_t = jnp.pad(emb_table, ((0, _VP - v), (0, 0))).T
    u16 = jax.lax.bitcast_convert_type(
        tab_t.astype(jnp.bfloat16), jnp.uint16
    ).astype(jnp.uint32)
    w01 = ((u16[:, 0:128] << 16) | u16[:, 128:256]).astype(jnp.int32)
    w23 = ((u16[:, 256:384] << 16) | u16[:, 384:512]).astype(jnp.int32)
    g_c = gamma.reshape(f, 1)
    b_c = beta.reshape(f, 1)
    bb = 4096 if b % 4096 == 0 else b
    lb = 4 if l % 4 == 0 else 1
    out_t = pl.pallas_call(
        functools.partial(_body, f=f, nb=bb, lb=lb, ch=256 if bb % 256 == 0 else bb),
        grid=(l // lb, b // bb),
        in_specs=[
            pl.BlockSpec((lb, f, bb), lambda i, j: (i, 0, j)),
            pl.BlockSpec((lb, 1, bb), lambda i, j: (i, 0, j)),
            pl.BlockSpec((f, 128), lambda i, j: (0, 0)),
            pl.BlockSpec((f, 128), lambda i, j: (0, 0)),
            pl.BlockSpec((f, 1), lambda i, j: (0, 0)),
            pl.BlockSpec((f, 1), lambda i, j: (0, 0)),
        ],
        out_specs=pl.BlockSpec((lb, f, bb), lambda i, j: (i, 0, j)),
        out_shape=jax.ShapeDtypeStruct((l, f, b), jnp.float32),
    )(xt, idx_t, w01, w23, g_c, b_c)
    return jnp.transpose(out_t, (2, 0, 1))


# speedup vs baseline: 7.7019x; 1.3471x over previous
import functools

import jax
import jax.numpy as jnp
from jax.experimental import pallas as pl

_EPS = 1e-12
_VP = 512


def _body(x_ref, idx_ref, w01_ref, w23_ref, g_ref, b_ref,
          o_ref, *, f, nb, lb, ch):
    w01_t = w01_ref[...]
    w23_t = w23_ref[...]
    g_c = g_ref[...]
    b_c = b_ref[...]
    for t in range(lb):
        idx2 = idx_ref[t]
        lo1 = idx2 & 127
        hi1 = idx2 >> 8
        sh1 = (idx2 & 128) >> 3
        for c in range(nb // ch):
            sl = pl.ds(c * ch, ch)
            cs = slice(c * ch, (c + 1) * ch)
            lo = jnp.broadcast_to(lo1[:, cs], (f, ch))
            w01 = jnp.take_along_axis(w01_t, lo, axis=1)
            w23 = jnp.take_along_axis(w23_t, lo, axis=1)
            w = jnp.where(hi1[:, cs] == 0, w01, w23)
            pe = jax.lax.bitcast_convert_type(w << sh1[:, cs], jnp.float32)
            h = x_ref[t, :, sl] + pe
            mean = jnp.mean(h, axis=0, keepdims=True)
            cdev = h - mean
            var = jnp.mean(cdev * cdev, axis=0, keepdims=True)
            r = jax.lax.rsqrt(var + _EPS)
            o_ref[t, :, sl] = (cdev * r) * g_c + b_c


def _pack_pairs(a, b):
    return jnp.int32(
        (a.astype(jnp.uint32) << 16) | b.astype(jnp.uint32)
    )


def kernel(x, position_tags, emb_table, gamma, beta):
    b, l, f = x.shape
    v = emb_table.shape[0]
    xt = jnp.transpose(x, (1, 2, 0))
    idx_t = position_tags.astype(jnp.int32).T.reshape(l, 1, b)
    tabp = jnp.pad(emb_table, ((0, _VP - v), (0, 0)))
    u16 = jax.lax.bitcast_convert_type(
        tabp.T.astype(jnp.bfloat16), jnp.uint16
    )
    w01 = _pack_pairs(u16[:, 0:128], u16[:, 128:256])
    w23 = _pack_pairs(u16[:, 256:384], u16[:, 384:512])
    g_c = gamma.reshape(f, 1)
    b_c = beta.reshape(f, 1)
    bb = 4096 if b % 4096 == 0 else b
    lb = 4 if l % 4 == 0 else 1
    ch = 128 if bb % 128 == 0 else bb
    out_t = pl.pallas_call(
        functools.partial(_body, f=f, nb=bb, lb=lb, ch=ch),
        grid=(l // lb, b // bb),
        in_specs=[
            pl.BlockSpec((lb, f, bb), lambda i, j: (i, 0, j)),
            pl.BlockSpec((lb, 1, bb), lambda i, j: (i, 0, j)),
            pl.BlockSpec((f, 128), lambda i, j: (0, 0)),
            pl.BlockSpec((f, 128), lambda i, j: (0, 0)),
            pl.BlockSpec((f, 1), lambda i, j: (0, 0)),
            pl.BlockSpec((f, 1), lambda i, j: (0, 0)),
        ],
        out_specs=pl.BlockSpec((lb, f, bb), lambda i, j: (i, 0, j)),
        out_shape=jax.ShapeDtypeStruct((l, f, b), jnp.float32),
    )(xt, idx_t, w01, w23, g_c, b_c)
    return jnp.transpose(out_t, (2, 0, 1))


# speedup vs baseline: 8.3179x; 1.0800x over previous
import functools

import jax
import jax.numpy as jnp
from jax.experimental import pallas as pl

_EPS = 1e-12
_VP = 512


def _body(x_ref, idx_ref, w01_ref, w23_ref, g_ref, b_ref,
          o_ref, *, f, nb, lb, ch):
    w01_t = w01_ref[...]
    w23_t = w23_ref[...]
    g_c = g_ref[...]
    b_c = b_ref[...]
    for t in range(lb):
        idx2 = idx_ref[t]
        lo1 = idx2 & 127
        hi1 = idx2 >> 8
        sh1 = (idx2 & 128) >> 3
        for c in range(nb // ch):
            sl = pl.ds(c * ch, ch)
            cs = slice(c * ch, (c + 1) * ch)
            lo = jnp.broadcast_to(lo1[:, cs], (f, ch))
            w01 = jnp.take_along_axis(w01_t, lo, axis=1)
            w23 = jnp.take_along_axis(w23_t, lo, axis=1)
            w = jnp.where(hi1[:, cs] == 0, w01, w23)
            pe = jax.lax.bitcast_convert_type(w << sh1[:, cs], jnp.float32)
            h = x_ref[t, :, sl] + pe
            mean = jnp.mean(h, axis=0, keepdims=True)
            cdev = h - mean
            var = jnp.mean(cdev * cdev, axis=0, keepdims=True)
            r = jax.lax.rsqrt(var + _EPS)
            o_ref[t, :, sl] = (cdev * r) * g_c + b_c


def _pack_pairs(a, b):
    return jnp.int32(
        (a.astype(jnp.uint32) << 16) | b.astype(jnp.uint32)
    )


def kernel(x, position_tags, emb_table, gamma, beta):
    b, l, f = x.shape
    v = emb_table.shape[0]
    xt = jnp.transpose(x, (1, 2, 0))
    idx_t = position_tags.astype(jnp.int32).T.reshape(l, 1, b)
    tabp = jnp.pad(emb_table, ((0, _VP - v), (0, 0)))
    u16 = jax.lax.bitcast_convert_type(
        tabp.T.astype(jnp.bfloat16), jnp.uint16
    )
    w01 = _pack_pairs(u16[:, 0:128], u16[:, 128:256])
    w23 = _pack_pairs(u16[:, 256:384], u16[:, 384:512])
    g_c = gamma.reshape(f, 1)
    b_c = beta.reshape(f, 1)
    bb = 4096 if b % 4096 == 0 else b
    lb = 8 if l % 8 == 0 else 1
    ch = 128 if bb % 128 == 0 else bb
    out_t = pl.pallas_call(
        functools.partial(_body, f=f, nb=bb, lb=lb, ch=ch),
        grid=(l // lb, b // bb),
        in_specs=[
            pl.BlockSpec((lb, f, bb), lambda i, j: (i, 0, j)),
            pl.BlockSpec((lb, 1, bb), lambda i, j: (i, 0, j)),
            pl.BlockSpec((f, 128), lambda i, j: (0, 0)),
            pl.BlockSpec((f, 128), lambda i, j: (0, 0)),
            pl.BlockSpec((f, 1), lambda i, j: (0, 0)),
            pl.BlockSpec((f, 1), lambda i, j: (0, 0)),
        ],
        out_specs=pl.BlockSpec((lb, f, bb), lambda i, j: (i, 0, j)),
        out_shape=jax.ShapeDtypeStruct((l, f, b), jnp.float32),
    )(xt, idx_t, w01, w23, g_c, b_c)
    return jnp.transpose(out_t, (2, 0, 1))
